# baseline (speedup 1.0000x reference)
import functools

import jax
import jax.numpy as jnp
from jax import lax
from jax.experimental import pallas as pl
from jax.experimental.pallas import tpu as pltpu
from jax.experimental.pallas import tpu_sc as plsc

_B = 16384
_E = 64
_TL = 20
_DL = 200
_NW = 32
_BG = 128
_BPW = 4
_CW = _BG * _BPW


def _do_table(idxT, tbl, out4, idx_v, g_v, t_v, sem, wid, n_words):
    col0 = wid * _CW

    @pl.loop(0, n_words)
    def _word(j):
        pltpu.sync_copy(idxT.at[j, pl.ds(col0, _CW)], idx_v)
        for b in range(_BPW):
            pltpu.async_copy(tbl.at[idx_v.at[pl.ds(b * _BG, _BG)]],
                             g_v.at[b], sem)
        for b in range(_BPW):
            pltpu.make_async_copy(tbl.at[idx_v.at[pl.ds(b * _BG, _BG)]],
                                  g_v.at[b], sem).wait()
            lanes = lax.iota(jnp.int32, 16)
            zero = jnp.zeros((16,), jnp.int32)

            @pl.loop(0, 8)
            def _eg(eg):
                for es in range(8):
                    e = eg * 8 + es
                    cols = zero + e
                    for s in range(8):
                        v = plsc.load_gather(g_v.at[b],
                                             [s * 16 + lanes, cols])
                        t_v[eg, es, pl.ds(s * 16, 16)] = v

            pltpu.sync_copy(
                t_v, out4.at[j, :, pl.ds((wid * _BPW + b) * 8, 8)])


def _body(t_idxT, d_idxT, t_tbl, d_tbl, out_t4, out_d4,
          idx_v, g_v, t_v, sem):
    wid = lax.axis_index("s") * 2 + lax.axis_index("c")
    _do_table(t_idxT, t_tbl, out_t4, idx_v, g_v, t_v, sem, wid, _TL)
    _do_table(d_idxT, d_tbl, out_d4, idx_v, g_v, t_v, sem, wid, _DL)


@jax.jit
def _lookup(t_idxT, d_idxT, t_tbl, d_tbl):
    mesh = plsc.VectorSubcoreMesh(core_axis_name="c", subcore_axis_name="s")
    return pl.kernel(
        _body,
        out_type=(
            jax.ShapeDtypeStruct((_TL, 8, 1024, 128), jnp.float32),
            jax.ShapeDtypeStruct((_DL, 8, 1024, 128), jnp.float32),
        ),
        mesh=mesh,
        scratch_types=[
            pltpu.VMEM((_CW,), jnp.int32),
            pltpu.VMEM((_BPW, _BG, _E), jnp.float32),
            pltpu.VMEM((8, 8, 128), jnp.float32),
            pltpu.SemaphoreType.DMA,
        ],
        compiler_params=pltpu.CompilerParams(use_tc_tiling_on_sc=False,
                                             needs_layout_passes=False),
    )(t_idxT, d_idxT, t_tbl, d_tbl)


def _detile(o4, n_words):
    o5 = o4.reshape(n_words, 8, 128, 8, 128)
    return o5.transpose(2, 4, 0, 1, 3).reshape(_B, n_words, _E)


def kernel(title, description, title_table, description_table):
    out_t4, out_d4 = _lookup(
        title.astype(jnp.int32).T,
        description.astype(jnp.int32).T,
        title_table,
        description_table,
    )
    return (_detile(out_t4, _TL), _detile(out_d4, _DL))


# speedup vs baseline: 1.0441x; 1.0441x over previous
import functools

import jax
import jax.numpy as jnp
from jax import lax
from jax.experimental import pallas as pl
from jax.experimental.pallas import tpu as pltpu
from jax.experimental.pallas import tpu_sc as plsc

_B = 16384
_E = 64
_TL = 20
_DL = 200
_NW = 32
_BG = 128
_BPW = 4
_CW = _BG * _BPW


def _do_table(idxT, tbl, out4, idx_v, g_v, t_v, sem, wsem, wid, n_words):
    col0 = wid * _CW
    lanes = lax.iota(jnp.int32, 16)
    zero = jnp.zeros((16,), jnp.int32)
    rows_s = [s * 16 + lanes for s in range(8)]

    def wb_dst(j, b):
        return out4.at[j, :, pl.ds((wid * _BPW + b) * 8, 8)]

    @pl.loop(0, n_words)
    def _word(j):
        pltpu.sync_copy(idxT.at[j, pl.ds(col0, _CW)], idx_v)
        for b in range(_BPW):
            pltpu.async_copy(tbl.at[idx_v.at[pl.ds(b * _BG, _BG)]],
                             g_v.at[b], sem)
        for b in range(_BPW):
            pltpu.make_async_copy(tbl.at[idx_v.at[pl.ds(b * _BG, _BG)]],
                                  g_v.at[b], sem).wait()

            @pl.when(j > 0)
            def _():
                pltpu.make_async_copy(t_v.at[b], wb_dst(0, b), wsem[b]).wait()

            @pl.loop(0, 8)
            def _eg(eg):
                for es in range(8):
                    cols = zero + (eg * 8 + es)
                    for s in range(8):
                        v = plsc.load_gather(g_v.at[b], [rows_s[s], cols])
                        t_v[b, eg, es, pl.ds(s * 16, 16)] = v

            pltpu.async_copy(t_v.at[b], wb_dst(j, b), wsem[b])

    for b in range(_BPW):
        pltpu.make_async_copy(t_v.at[b], wb_dst(0, b), wsem[b]).wait()


def _body(t_idxT, d_idxT, t_tbl, d_tbl, out_t4, out_d4,
          idx_v, g_v, t_v, sem, w0, w1, w2, w3):
    wid = lax.axis_index("s") * 2 + lax.axis_index("c")
    wsem = (w0, w1, w2, w3)
    _do_table(t_idxT, t_tbl, out_t4, idx_v, g_v, t_v, sem, wsem, wid, _TL)
    _do_table(d_idxT, d_tbl, out_d4, idx_v, g_v, t_v, sem, wsem, wid, _DL)


@jax.jit
def _lookup(t_idxT, d_idxT, t_tbl, d_tbl):
    mesh = plsc.VectorSubcoreMesh(core_axis_name="c", subcore_axis_name="s")
    return pl.kernel(
        _body,
        out_type=(
            jax.ShapeDtypeStruct((_TL, 8, 1024, 128), jnp.float32),
            jax.ShapeDtypeStruct((_DL, 8, 1024, 128), jnp.float32),
        ),
        mesh=mesh,
        scratch_types=[
            pltpu.VMEM((_CW,), jnp.int32),
            pltpu.VMEM((_BPW, _BG, _E), jnp.float32),
            pltpu.VMEM((_BPW, 8, 8, 128), jnp.float32),
            pltpu.SemaphoreType.DMA,
            pltpu.SemaphoreType.DMA,
            pltpu.SemaphoreType.DMA,
            pltpu.SemaphoreType.DMA,
            pltpu.SemaphoreType.DMA,
        ],
        compiler_params=pltpu.CompilerParams(use_tc_tiling_on_sc=False,
                                             needs_layout_passes=False),
    )(t_idxT, d_idxT, t_tbl, d_tbl)


def _detile(o4, n_words):
    o5 = o4.reshape(n_words, 8, 128, 8, 128)
    return o5.transpose(2, 4, 0, 1, 3).reshape(_B, n_words, _E)


def kernel(title, description, title_table, description_table):
    out_t4, out_d4 = _lookup(
        title.astype(jnp.int32).T,
        description.astype(jnp.int32).T,
        title_table,
        description_table,
    )
    return (_detile(out_t4, _TL), _detile(out_d4, _DL))


# speedup vs baseline: 2.1427x; 2.0522x over previous
import functools

import jax
import jax.numpy as jnp
from jax import lax
from jax.experimental import pallas as pl
from jax.experimental.pallas import tpu as pltpu
from jax.experimental.pallas import tpu_sc as plsc

_EMB = 64
_B = 16384
_TL = 20
_DL = 200

_NC = 2
_NS = 16
_NW = _NC * _NS

_IW = 128
_KPC = 4
_CHUNK = _KPC * _IW
_NBUF = 2

_T_TOTAL = _B * _TL
_D_TOTAL = _B * _DL
_T_ROWS = _T_TOTAL // _IW
_D_ROWS = _D_TOTAL // _IW
_T_ROWS_W = _T_ROWS // _NW
_D_ROWS_W = _D_ROWS // _NW


def _gather_table(tbl, idx_hbm, out_hbm, idx_v, rows_v, sem_i, sem_g, sem_w,
                  wid, rows_w):
    base = wid * rows_w
    n_chunks = rows_w // _KPC
    n_groups = n_chunks // _NBUF

    def idx_src(c):
        return idx_hbm.at[pl.ds(base + c * _KPC, _KPC)]

    def out_dst(c):
        return out_hbm.at[pl.ds((base + c * _KPC) * _IW, _CHUNK)]

    def fire_gathers(b):
        for j in range(_KPC):
            pltpu.async_copy(tbl.at[idx_v.at[b, j]],
                             rows_v.at[b, pl.ds(j * _IW, _IW)], sem_g[b])

    def wait_gathers(b):
        for j in range(_KPC):
            pltpu.make_async_copy(tbl.at[idx_v.at[b, j]],
                                  rows_v.at[b, pl.ds(j * _IW, _IW)],
                                  sem_g[b]).wait()

    def wait_idx(b):
        pltpu.make_async_copy(idx_src(0), idx_v.at[b], sem_i[b]).wait()

    def wait_wb(b):
        pltpu.make_async_copy(rows_v.at[b], out_dst(0), sem_w[b]).wait()

    for b in range(_NBUF):
        pltpu.async_copy(idx_src(b), idx_v.at[b], sem_i[b])

    @pl.loop(0, n_groups)
    def _group(gi):
        c0 = gi * _NBUF
        for b in range(_NBUF):
            wait_idx(b)

            @pl.when(gi > 0)
            def _():
                wait_wb(b)

            fire_gathers(b)
        for b in range(_NBUF):
            wait_gathers(b)
            pltpu.async_copy(rows_v.at[b], out_dst(c0 + b), sem_w[b])

            @pl.when(gi < n_groups - 1)
            def _():
                pltpu.async_copy(idx_src(c0 + _NBUF + b), idx_v.at[b],
                                 sem_i[b])

    for b in range(_NBUF):
        wait_wb(b)


def _sc_body(t_idx, d_idx, t_tbl, d_tbl, out_t, out_d, idx_v, rows_v,
             si0, si1, sg0, sg1, sw0, sw1):
    wid = lax.axis_index("s") * _NC + lax.axis_index("c")
    sem_i = (si0, si1)
    sem_g = (sg0, sg1)
    sem_w = (sw0, sw1)
    _gather_table(t_tbl, t_idx, out_t, idx_v, rows_v, sem_i, sem_g, sem_w,
                  wid, _T_ROWS_W)
    _gather_table(d_tbl, d_idx, out_d, idx_v, rows_v, sem_i, sem_g, sem_w,
                  wid, _D_ROWS_W)


def _sc_gather(t_idx, d_idx, t_tbl, d_tbl):
    mesh = plsc.VectorSubcoreMesh(core_axis_name="c", subcore_axis_name="s")
    return pl.kernel(
        _sc_body,
        out_type=(
            jax.ShapeDtypeStruct((_T_TOTAL, _EMB), jnp.float32),
            jax.ShapeDtypeStruct((_D_TOTAL, _EMB), jnp.float32),
        ),
        mesh=mesh,
        scratch_types=[
            pltpu.VMEM((_NBUF, _KPC, _IW), jnp.int32),
            pltpu.VMEM((_NBUF, _CHUNK, _EMB), jnp.float32),
        ] + [pltpu.SemaphoreType.DMA] * 6,
        compiler_params=pltpu.CompilerParams(use_tc_tiling_on_sc=False),
    )(t_idx, d_idx, t_tbl, d_tbl)


def _tc_body(n_words, x_ref, o_ref):
    for j2 in range(n_words // 2):
        x = x_ref[0, :, j2, :]
        o_ref[pl.ds(2 * j2, 2)] = x.T.reshape(2, 8, 8, 128)


def _tc_detile(flat, n_words):
    x4 = flat.reshape(_B // 128, 128, n_words // 2, 128)
    return pl.pallas_call(
        functools.partial(_tc_body, n_words),
        grid=(_B // 128,),
        in_specs=[pl.BlockSpec((1, 128, n_words // 2, 128),
                               lambda bg: (bg, 0, 0, 0))],
        out_specs=pl.BlockSpec((n_words, 8, 8, 128),
                               lambda bg: (0, 0, bg, 0)),
        out_shape=jax.ShapeDtypeStruct((n_words, 8, 1024, 128), jnp.float32),
    )(x4)


def _relabel(o4, n_words):
    o5 = o4.reshape(n_words, 8, 128, 8, 128)
    return o5.transpose(2, 4, 0, 1, 3).reshape(_B, n_words, _EMB)


@jax.jit
def _lookup(title, description, title_table, description_table):
    t_idx = title.reshape(_T_ROWS, _IW).astype(jnp.int32)
    d_idx = description.reshape(_D_ROWS, _IW).astype(jnp.int32)
    out_t, out_d = _sc_gather(t_idx, d_idx, title_table, description_table)
    o4_t = _tc_detile(out_t.reshape(_T_TOTAL // 2, 2 * _EMB), _TL)
    o4_d = _tc_detile(out_d.reshape(_D_TOTAL // 2, 2 * _EMB), _DL)
    return (_relabel(o4_t, _TL), _relabel(o4_d, _DL))


def kernel(title, description, title_table, description_table):
    return _lookup(title, description, title_table, description_table)
